# baseline (speedup 1.0000x reference)
import functools

import jax
import jax.numpy as jnp
from jax import lax
from jax.experimental import pallas as pl
from jax.experimental.pallas import tpu as pltpu
from jax.experimental.pallas import tpu_sc as plsc

_NC = 2
_NS = 16
_NW = _NC * _NS
_L = 16

_B = 4096
_F = 100
_FPAD = 112
_D = 128
_DV = _D // _L
_BPW = _B // _NW
_NBUF = 4


def _fm_body(idx_hbm, wl_hbm, bias_hbm, v_hbm, out_hbm,
             idx_all, out_v, bias_v, *bufs):
    rows = bufs[0:_NBUF]
    wvs = bufs[_NBUF:2 * _NBUF]
    sem_r = bufs[2 * _NBUF:3 * _NBUF]
    sem_w = bufs[3 * _NBUF:4 * _NBUF]

    wid = lax.axis_index("s") * _NC + lax.axis_index("c")
    base = wid * _BPW

    pltpu.sync_copy(idx_hbm.at[pl.ds(base, _BPW)], idx_all)
    pltpu.sync_copy(bias_hbm, bias_v.at[pl.ds(0, 1)])
    bias_s = bias_v[pl.ds(0, _L)][0]

    zero16 = jnp.zeros((_L,), jnp.float32)
    for w in wvs:
        for c in range(_FPAD // _L):
            w[pl.ds(c * _L, _L)] = zero16

    _FA = 48
    _FB = _F - _FA

    def start(b, slot):
        idx_a = idx_all.at[b, pl.ds(0, _FA)]
        idx_b = idx_all.at[b, pl.ds(_FA, _FB)]
        pltpu.async_copy(v_hbm.at[idx_a], rows[slot].at[pl.ds(0, _FA)],
                         sem_r[slot])
        pltpu.async_copy(v_hbm.at[idx_b], rows[slot].at[pl.ds(_FA, _FB)],
                         sem_w[slot])
        pltpu.async_copy(wl_hbm.at[idx_all.at[b]],
                         wvs[slot].at[pl.ds(0, _F)], sem_w[slot])

    def compute(b, slot):
        idx_a = idx_all.at[b, pl.ds(0, _FA)]
        idx_b = idx_all.at[b, pl.ds(_FA, _FB)]
        pltpu.make_async_copy(v_hbm.at[idx_a],
                              rows[slot].at[pl.ds(0, _FA)],
                              sem_r[slot]).wait()
        pltpu.make_async_copy(v_hbm.at[idx_b],
                              rows[slot].at[pl.ds(_FA, _FB)],
                              sem_w[slot]).wait()
        pltpu.make_async_copy(wl_hbm.at[idx_all.at[b]],
                              wvs[slot].at[pl.ds(0, _F)],
                              sem_w[slot]).wait()
        r_ref = rows[slot]

        def row_body(r, carry):
            s = carry[:_DV]
            q = carry[_DV:]
            new_s = []
            new_q = []
            for j in range(_DV):
                v = r_ref[r, pl.ds(j * _L, _L)]
                new_s.append(s[j] + v)
                new_q.append(q[j] + v * v)
            return tuple(new_s) + tuple(new_q)

        carry = lax.fori_loop(0, _F, row_body, (zero16,) * (2 * _DV),
                              unroll=4)

        acc = zero16
        for j in range(_DV):
            s_j = carry[j]
            q_j = carry[_DV + j]
            acc = acc + (s_j * s_j - q_j)
        fm = jnp.sum(acc)

        w_ref = wvs[slot]
        lv = zero16
        for c in range(_FPAD // _L):
            lv = lv + w_ref[pl.ds(c * _L, _L)]
        lin = jnp.sum(lv)

        return lin + bias_s + 0.5 * fm

    for s in range(_NBUF):
        start(s, s)
    lane = lax.iota(jnp.int32, _L)

    def outer(bg, y_vec):
        b0 = _NBUF * bg
        for s in range(_NBUF):
            b = b0 + s
            y = compute(b, s)

            @pl.when(b + _NBUF < _BPW)
            def _prefetch():
                start(b + _NBUF, s)

            y_vec = jnp.where(lane == b % _L, jnp.full((_L,), y), y_vec)

        @pl.when((b0 + _NBUF) % _L == 0)
        def _flush():
            out_v[pl.ds((b0 // _L) * _L, _L)] = y_vec

        return y_vec

    lax.fori_loop(0, _BPW // _NBUF, outer, zero16)

    pltpu.sync_copy(out_v, out_hbm.at[pl.ds(base, _BPW)])


_fm = functools.partial(
    pl.kernel,
    out_type=jax.ShapeDtypeStruct((_B,), jnp.float32),
    mesh=plsc.VectorSubcoreMesh(core_axis_name="c", subcore_axis_name="s",
                                num_cores=_NC, num_subcores=_NS),
    compiler_params=pltpu.CompilerParams(needs_layout_passes=False),
    scratch_types=[
        pltpu.VMEM((_BPW, _F), jnp.int32),
        pltpu.VMEM((_BPW,), jnp.float32),
        pltpu.VMEM((1,), jnp.float32),
    ]
    + [pltpu.VMEM((_F, _D), jnp.float32) for _ in range(_NBUF)]
    + [pltpu.VMEM((_FPAD,), jnp.float32) for _ in range(_NBUF)]
    + [pltpu.SemaphoreType.DMA for _ in range(2 * _NBUF)],
)(_fm_body)


@jax.jit
def kernel(x, W_lin, bias, V):
    n_fields = x.shape[1]
    rows_per_field = V.shape[0] // n_fields
    offsets = (jnp.arange(n_fields, dtype=jnp.int32) * rows_per_field)
    idx = x.astype(jnp.int32) + offsets[None, :]
    return _fm(idx, W_lin.reshape(-1), bias, V)


# speedup vs baseline: 1.0204x; 1.0204x over previous
import functools

import jax
import jax.numpy as jnp
from jax import lax
from jax.experimental import pallas as pl
from jax.experimental.pallas import tpu as pltpu
from jax.experimental.pallas import tpu_sc as plsc

_NC = 2
_NS = 16
_NW = _NC * _NS
_L = 16

_B = 4096
_F = 100
_FPAD = 112
_D = 128
_DV = _D // _L
_BPW = _B // _NW
_NBUF = 4
_RPF = 1000


def _fm_body(idx_hbm, wl_hbm, bias_hbm, v_hbm, out_hbm,
             idx_all, out_v, bias_v, *bufs):
    rows = bufs[0:_NBUF]
    wvs = bufs[_NBUF:2 * _NBUF]
    sem_r = bufs[2 * _NBUF:3 * _NBUF]
    sem_w = bufs[3 * _NBUF:4 * _NBUF]

    wid = lax.axis_index("s") * _NC + lax.axis_index("c")
    base = wid * _BPW

    pltpu.sync_copy(idx_hbm.at[pl.ds(base, _BPW)], idx_all)
    pltpu.sync_copy(bias_hbm, bias_v.at[pl.ds(0, 1)])
    bias_s = bias_v[pl.ds(0, _L)][0]
    lane = lax.iota(jnp.int32, _L)
    lane_off = lane * _RPF

    zero16 = jnp.zeros((_L,), jnp.float32)
    for w in wvs:
        for c in range(_FPAD // _L):
            w[pl.ds(c * _L, _L)] = zero16

    _TAIL = _F - _L

    def start(b, slot):
        raw_tail = idx_all[b, pl.ds(_TAIL, _L)]
        for c in range(_F // _L):
            t = idx_all[b, pl.ds(c * _L, _L)]
            idx_all[b, pl.ds(c * _L, _L)] = t + (c * _L * _RPF + lane_off)
        idx_all[b, pl.ds(_TAIL, _L)] = raw_tail + (_TAIL * _RPF + lane_off)
        idx_row = idx_all.at[b]
        pltpu.async_copy(v_hbm.at[idx_row], rows[slot], sem_r[slot])
        pltpu.async_copy(wl_hbm.at[idx_row], wvs[slot].at[pl.ds(0, _F)],
                         sem_w[slot])

    def compute(b, slot):
        idx_row = idx_all.at[b]
        pltpu.make_async_copy(v_hbm.at[idx_row], rows[slot],
                              sem_r[slot]).wait()
        pltpu.make_async_copy(wl_hbm.at[idx_row],
                              wvs[slot].at[pl.ds(0, _F)],
                              sem_w[slot]).wait()
        r_ref = rows[slot]

        def row_body(r, carry):
            s = carry[:_DV]
            q = carry[_DV:]
            new_s = []
            new_q = []
            for j in range(_DV):
                v = r_ref[r, pl.ds(j * _L, _L)]
                new_s.append(s[j] + v)
                new_q.append(q[j] + v * v)
            return tuple(new_s) + tuple(new_q)

        carry = lax.fori_loop(0, _F, row_body, (zero16,) * (2 * _DV),
                              unroll=4)

        acc = zero16
        for j in range(_DV):
            s_j = carry[j]
            q_j = carry[_DV + j]
            acc = acc + (s_j * s_j - q_j)
        fm = jnp.sum(acc)

        w_ref = wvs[slot]
        lv = zero16
        for c in range(_FPAD // _L):
            lv = lv + w_ref[pl.ds(c * _L, _L)]
        lin = jnp.sum(lv)

        return lin + bias_s + 0.5 * fm

    for s in range(_NBUF):
        start(s, s)

    def outer(bg, y_vec):
        b0 = _NBUF * bg
        for s in range(_NBUF):
            b = b0 + s
            y = compute(b, s)

            @pl.when(b + _NBUF < _BPW)
            def _prefetch():
                start(b + _NBUF, s)

            y_vec = jnp.where(lane == b % _L, jnp.full((_L,), y), y_vec)

        @pl.when((b0 + _NBUF) % _L == 0)
        def _flush():
            out_v[pl.ds((b0 // _L) * _L, _L)] = y_vec

        return y_vec

    lax.fori_loop(0, _BPW // _NBUF, outer, zero16)

    pltpu.sync_copy(out_v, out_hbm.at[pl.ds(base, _BPW)])


_fm = functools.partial(
    pl.kernel,
    out_type=jax.ShapeDtypeStruct((_B,), jnp.float32),
    mesh=plsc.VectorSubcoreMesh(core_axis_name="c", subcore_axis_name="s",
                                num_cores=_NC, num_subcores=_NS),
    compiler_params=pltpu.CompilerParams(needs_layout_passes=False),
    scratch_types=[
        pltpu.VMEM((_BPW, _F), jnp.int32),
        pltpu.VMEM((_BPW,), jnp.float32),
        pltpu.VMEM((1,), jnp.float32),
    ]
    + [pltpu.VMEM((_F, _D), jnp.float32) for _ in range(_NBUF)]
    + [pltpu.VMEM((_FPAD,), jnp.float32) for _ in range(_NBUF)]
    + [pltpu.SemaphoreType.DMA for _ in range(2 * _NBUF)],
)(_fm_body)


@jax.jit
def kernel(x, W_lin, bias, V):
    return _fm(x.astype(jnp.int32), W_lin.reshape(-1), bias, V)
